# baseline (speedup 1.0000x reference)
import dataclasses
import functools

import jax
import jax.numpy as jnp
from jax import lax
from jax.experimental import pallas as pl
from jax.experimental.pallas import tpu as pltpu
from jax.experimental.pallas import tpu_sc as plsc

_NC = 2
_NS = 16
_L = 16
_KB = 32


def kernel(user, item, user_emb, item_emb):
    B = user.shape[0]
    K, N = item_emb.shape
    M = user_emb.shape[0]
    NW = _NC * _NS
    CH = B // NW
    NKB = K // _KB

    uflat = user_emb.T.reshape(K * M)
    iflat = item_emb.reshape(K * N)
    mesh = plsc.VectorSubcoreMesh(core_axis_name="c", subcore_axis_name="s")
    cp = pltpu.CompilerParams()
    if "needs_layout_passes" in pltpu.CompilerParams.__dataclass_fields__:
        cp = dataclasses.replace(cp, needs_layout_passes=False)
    if "use_tc_tiling_on_sc" in pltpu.CompilerParams.__dataclass_fields__:
        cp = dataclasses.replace(cp, use_tc_tiling_on_sc=True)

    @functools.partial(
        pl.kernel,
        out_type=jax.ShapeDtypeStruct((B,), jnp.float32),
        mesh=mesh,
        compiler_params=cp,
        scratch_types=[
            pltpu.VMEM((CH,), jnp.int32),
            pltpu.VMEM((CH,), jnp.int32),
            pltpu.VMEM((_KB * CH,), jnp.int32),
            pltpu.VMEM((_KB * CH,), jnp.int32),
            pltpu.VMEM((_KB * CH,), jnp.float32),
            pltpu.VMEM((_KB * CH,), jnp.float32),
            pltpu.VMEM((CH,), jnp.float32),
            pltpu.SemaphoreType.DMA,
            pltpu.SemaphoreType.DMA,
        ],
    )
    def sc_kernel(user_hbm, item_hbm, uflat_hbm, iflat_hbm, out_hbm,
                  uidx_v, iidx_v, gu_v, gi_v, u_v, v_v, acc_v, sem_u, sem_v):
        wid = lax.axis_index("s") * _NC + lax.axis_index("c")
        base = wid * CH
        pltpu.sync_copy(user_hbm.at[pl.ds(base, CH)], uidx_v)
        pltpu.sync_copy(item_hbm.at[pl.ds(base, CH)], iidx_v)

        for c in range(CH // _L):
            acc_v[pl.ds(c * _L, _L)] = jnp.zeros((_L,), jnp.float32)

        @pl.loop(0, NKB)
        def _kblock(kb):
            k0 = kb * _KB

            @pl.loop(0, _KB)
            def _build(kk):
                off_u = (k0 + kk) * M
                off_i = (k0 + kk) * N
                for c in range(CH // _L):
                    sl = pl.ds(kk * CH + c * _L, _L)
                    src = pl.ds(c * _L, _L)
                    gu_v[sl] = uidx_v[src] + off_u
                    gi_v[sl] = iidx_v[src] + off_i

            cp_u = pltpu.async_copy(uflat_hbm.at[gu_v], u_v, sem_u)
            cp_v = pltpu.async_copy(iflat_hbm.at[gi_v], v_v, sem_v)
            cp_u.wait()
            cp_v.wait()

            @pl.loop(0, _KB)
            def _mac(kk):
                for c in range(CH // _L):
                    sl = pl.ds(kk * CH + c * _L, _L)
                    dst = pl.ds(c * _L, _L)
                    acc_v[dst] = acc_v[dst] + u_v[sl] * v_v[sl]

        @pl.loop(0, CH, step=_L)
        def _fin(c):
            pre = jnp.maximum(4.0 - acc_v[pl.ds(c, _L)], 0.0)
            pre = jnp.maximum(4.0 - pre, 0.0) + 1.0
            acc_v[pl.ds(c, _L)] = pre

        pltpu.sync_copy(acc_v, out_hbm.at[pl.ds(base, CH)])

    return sc_kernel(user, item, uflat, iflat)


# speedup vs baseline: 16.7159x; 16.7159x over previous
import dataclasses
import functools

import jax
import jax.numpy as jnp
from jax import lax
from jax.experimental import pallas as pl
from jax.experimental.pallas import tpu as pltpu
from jax.experimental.pallas import tpu_sc as plsc

_NC = 2
_NS = 16
_L = 16
_KG = 16
_W = 16384


def _detile_group(table, g, kg, w):
    K, R = table.shape

    def body(in_ref, *out_refs):
        for j in range(kg):
            out_refs[j][...] = in_ref[j, :]

    grid = (pl.cdiv(R, w),)
    return pl.pallas_call(
        body,
        grid=grid,
        in_specs=[pl.BlockSpec((kg, w), lambda s, g=g: (g, s))],
        out_specs=[pl.BlockSpec((w,), lambda s: (s,)) for _ in range(kg)],
        out_shape=[jax.ShapeDtypeStruct((R,), table.dtype) for _ in range(kg)],
    )(table)


def kernel(user, item, user_emb, item_emb):
    B = user.shape[0]
    K, N = item_emb.shape
    M = user_emb.shape[0]
    NW = _NC * _NS
    CH = B // NW
    G = K // _KG

    uT = user_emb.T
    mesh = plsc.VectorSubcoreMesh(core_axis_name="c", subcore_axis_name="s")
    cp = pltpu.CompilerParams()
    if "needs_layout_passes" in pltpu.CompilerParams.__dataclass_fields__:
        cp = dataclasses.replace(cp, needs_layout_passes=False)
    if "use_tc_tiling_on_sc" in pltpu.CompilerParams.__dataclass_fields__:
        cp = dataclasses.replace(cp, use_tc_tiling_on_sc=True)

    def make_gather(g):
        has_prev = g > 0
        is_last = g == G - 1
        n_in = 2 + (1 if has_prev else 0) + 2 * _KG

        scratch = [
            pltpu.VMEM((CH,), jnp.int32),
            pltpu.VMEM((CH,), jnp.int32),
            pltpu.VMEM((_KG * CH,), jnp.float32),
            pltpu.VMEM((_KG * CH,), jnp.float32),
            pltpu.VMEM((CH,), jnp.float32),
            pltpu.SemaphoreType.DMA,
            pltpu.SemaphoreType.DMA,
        ]
        if has_prev:
            scratch.append(pltpu.VMEM((CH,), jnp.float32))

        @functools.partial(
            pl.kernel,
            out_type=jax.ShapeDtypeStruct((B,), jnp.float32),
            mesh=mesh,
            compiler_params=cp,
            scratch_types=scratch,
        )
        def sc_gather(*refs):
            user_hbm, item_hbm = refs[0], refs[1]
            pos = 2
            prev_hbm = refs[pos] if has_prev else None
            pos += 1 if has_prev else 0
            utabs = refs[pos:pos + _KG]
            itabs = refs[pos + _KG:pos + 2 * _KG]
            out_hbm = refs[n_in]
            uidx_v, iidx_v, u_v, v_v, acc_v, sem_u, sem_v = refs[n_in + 1:n_in + 8]
            pbuf_v = refs[n_in + 8] if has_prev else None

            wid = lax.axis_index("s") * _NC + lax.axis_index("c")
            base = wid * CH
            pltpu.sync_copy(user_hbm.at[pl.ds(base, CH)], uidx_v)
            pltpu.sync_copy(item_hbm.at[pl.ds(base, CH)], iidx_v)

            cps = []
            for j in range(_KG):
                cps.append(pltpu.async_copy(
                    utabs[j].at[uidx_v], u_v.at[pl.ds(j * CH, CH)], sem_u))
                cps.append(pltpu.async_copy(
                    itabs[j].at[iidx_v], v_v.at[pl.ds(j * CH, CH)], sem_v))
            if has_prev:
                pltpu.sync_copy(prev_hbm.at[pl.ds(base, CH)], pbuf_v)
            for cpo in cps:
                cpo.wait()

            @pl.loop(0, CH, step=_L)
            def _mac(c):
                acc = (pbuf_v[pl.ds(c, _L)] if has_prev
                       else jnp.zeros((_L,), jnp.float32))
                for j in range(_KG):
                    acc = acc + u_v[pl.ds(j * CH + c, _L)] * v_v[pl.ds(j * CH + c, _L)]
                if is_last:
                    acc = jnp.maximum(4.0 - acc, 0.0)
                    acc = jnp.maximum(4.0 - acc, 0.0) + 1.0
                acc_v[pl.ds(c, _L)] = acc

            pltpu.sync_copy(acc_v, out_hbm.at[pl.ds(base, CH)])

        return sc_gather

    partial_out = None
    for g in range(G):
        utabs = _detile_group(uT, g, _KG, _W)
        itabs = _detile_group(item_emb, g, _KG, _W)
        args = [user, item]
        if partial_out is not None:
            args.append(partial_out)
        args.extend(utabs)
        args.extend(itabs)
        partial_out = make_gather(g)(*args)
    return partial_out


# speedup vs baseline: 23.3514x; 1.3970x over previous
import dataclasses
import functools

import jax
import jax.numpy as jnp
from jax import lax
from jax.experimental import pallas as pl
from jax.experimental.pallas import tpu as pltpu
from jax.experimental.pallas import tpu_sc as plsc

_NC = 2
_NS = 16
_L = 16
_W = 16384
_PW = 8192
_NB = 4


def _tc_detile(table, w):
    K, R = table.shape

    def body(in_ref, *out_refs):
        for j in range(K):
            out_refs[j][...] = in_ref[j, :]

    return pl.pallas_call(
        body,
        grid=(pl.cdiv(R, w),),
        in_specs=[pl.BlockSpec((K, w), lambda s: (0, s))],
        out_specs=[pl.BlockSpec((w,), lambda s: (s,)) for _ in range(K)],
        out_shape=[jax.ShapeDtypeStruct((R,), table.dtype) for _ in range(K)],
    )(table)


def kernel(user, item, user_emb, item_emb):
    B = user.shape[0]
    K, N = item_emb.shape
    M = user_emb.shape[0]
    NW = _NC * _NS
    CH = B // NW

    uT = user_emb.T
    mesh = plsc.VectorSubcoreMesh(core_axis_name="c", subcore_axis_name="s")
    cp = pltpu.CompilerParams()
    if "needs_layout_passes" in pltpu.CompilerParams.__dataclass_fields__:
        cp = dataclasses.replace(cp, needs_layout_passes=False)
    if "use_tc_tiling_on_sc" in pltpu.CompilerParams.__dataclass_fields__:
        cp = dataclasses.replace(cp, use_tc_tiling_on_sc=True)

    nfull = (N // _PW) * _PW
    nchunk = N // _PW
    tail = N - nfull

    @functools.partial(
        pl.kernel,
        out_type=jax.ShapeDtypeStruct((K * N,), jnp.float32),
        mesh=mesh,
        compiler_params=cp,
        scratch_types=[
            pltpu.VMEM((_NB * _PW,), jnp.float32),
            pltpu.VMEM((_NB * _PW,), jnp.float32),
            pltpu.VMEM((tail,), jnp.float32),
            pltpu.SemaphoreType.DMA,
            pltpu.SemaphoreType.DMA,
            pltpu.SemaphoreType.DMA,
        ],
    )
    def sc_detile(tab_hbm, out_hbm, buf_a, buf_b, tail_v, sem_a, sem_b, sem_t):
        wid = lax.axis_index("s") * _NC + lax.axis_index("c")
        rows_per_w = K // NW

        for r in range(rows_per_w):
            kk = wid * rows_per_w + r

            def read_batch(b0, nb, buf, sem):
                cps = []
                for q in range(nb):
                    cps.append(pltpu.async_copy(
                        tab_hbm.at[kk].at[pl.ds((b0 + q) * _PW, _PW)],
                        buf.at[pl.ds(q * _PW, _PW)], sem))
                return cps

            def write_batch(b0, nb, buf, sem):
                cps = []
                for q in range(nb):
                    cps.append(pltpu.async_copy(
                        buf.at[pl.ds(q * _PW, _PW)],
                        out_hbm.at[pl.ds(kk * N + (b0 + q) * _PW, _PW)], sem))
                return cps

            nbat = nchunk // _NB
            rem = nchunk - nbat * _NB
            prev_writes = []
            cur_reads = read_batch(0, _NB, buf_a, sem_a)
            for b in range(nbat):
                nxt = b + 1
                use_a = (b % 2) == 0
                buf = buf_a if use_a else buf_b
                nbuf = buf_b if use_a else buf_a
                nsem = sem_b if use_a else sem_a
                for c in cur_reads:
                    c.wait()
                if nxt < nbat:
                    nxt_reads = read_batch(nxt * _NB, _NB, nbuf, nsem)
                elif rem > 0:
                    nxt_reads = read_batch(nbat * _NB, rem, nbuf, nsem)
                else:
                    nxt_reads = []
                for c in prev_writes:
                    c.wait()
                prev_writes = write_batch(b * _NB, _NB, buf,
                                          sem_a if use_a else sem_b)
                cur_reads = nxt_reads
            if rem > 0:
                buf = buf_a if (nbat % 2) == 0 else buf_b
                for c in cur_reads:
                    c.wait()
                for c in prev_writes:
                    c.wait()
                prev_writes = write_batch(nbat * _NB, rem, buf,
                                          sem_a if (nbat % 2) == 0 else sem_b)
            if tail > 0:
                pltpu.async_copy(
                    tab_hbm.at[kk].at[pl.ds(nfull, tail)], tail_v, sem_t).wait()
                for c in prev_writes:
                    c.wait()
                pltpu.async_copy(
                    tail_v, out_hbm.at[pl.ds(kk * N + nfull, tail)], sem_t).wait()
            else:
                for c in prev_writes:
                    c.wait()

    @functools.partial(
        pl.kernel,
        out_type=jax.ShapeDtypeStruct((B,), jnp.float32),
        mesh=mesh,
        compiler_params=cp,
        scratch_types=[
            pltpu.VMEM((CH,), jnp.int32),
            pltpu.VMEM((CH,), jnp.int32),
            pltpu.VMEM((K * CH,), jnp.int32),
            pltpu.VMEM((K * CH,), jnp.float32),
            pltpu.VMEM((K * CH,), jnp.float32),
            pltpu.VMEM((CH,), jnp.float32),
            pltpu.SemaphoreType.DMA,
            pltpu.SemaphoreType.DMA,
        ],
    )
    def sc_gather(*refs):
        user_hbm, item_hbm, iflat_hbm = refs[0], refs[1], refs[2]
        utabs = refs[3:3 + K]
        out_hbm = refs[3 + K]
        (uidx_v, iidx_v, gidx_v, u_v, v_v, acc_v,
         sem_u, sem_v) = refs[4 + K:12 + K]

        wid = lax.axis_index("s") * _NC + lax.axis_index("c")
        base = wid * CH
        pltpu.sync_copy(user_hbm.at[pl.ds(base, CH)], uidx_v)
        pltpu.sync_copy(item_hbm.at[pl.ds(base, CH)], iidx_v)

        cps = []
        for j in range(K):
            cps.append(pltpu.async_copy(
                utabs[j].at[uidx_v], u_v.at[pl.ds(j * CH, CH)], sem_u))

        @pl.loop(0, K)
        def _build(kk):
            off = kk * N
            for c in range(CH // _L):
                gidx_v[pl.ds(kk * CH + c * _L, _L)] = (
                    iidx_v[pl.ds(c * _L, _L)] + off)

        cp_v = pltpu.async_copy(iflat_hbm.at[gidx_v], v_v, sem_v)
        for c in cps:
            c.wait()
        cp_v.wait()

        @pl.loop(0, CH, step=_L)
        def _mac(c):
            acc = jnp.zeros((_L,), jnp.float32)
            for j in range(K):
                acc = acc + u_v[pl.ds(j * CH + c, _L)] * v_v[pl.ds(j * CH + c, _L)]
            acc = jnp.maximum(4.0 - acc, 0.0)
            acc = jnp.maximum(4.0 - acc, 0.0) + 1.0
            acc_v[pl.ds(c, _L)] = acc

        pltpu.sync_copy(acc_v, out_hbm.at[pl.ds(base, CH)])

    utabs = _tc_detile(uT, _W)
    iflat = sc_detile(item_emb)
    return sc_gather(user, item, iflat, *utabs)


# speedup vs baseline: 24.0967x; 1.0319x over previous
import dataclasses
import functools

import jax
import jax.numpy as jnp
from jax import lax
from jax.experimental import pallas as pl
from jax.experimental.pallas import tpu as pltpu
from jax.experimental.pallas import tpu_sc as plsc

_NC = 2
_NS = 16
_L = 16
_W = 16384
_PW = 8192
_NB = 6
_KSC = 32


def _tc_detile(utable, itable, ksc, w):
    K, M = utable.shape
    _, N = itable.shape
    kt = K - ksc

    def body(u_ref, i_ref, *out_refs):
        for j in range(K):
            out_refs[j][...] = u_ref[j, :]
        for j in range(kt):
            out_refs[K + j][...] = i_ref[j, :]

    return pl.pallas_call(
        body,
        grid=(pl.cdiv(N, w),),
        in_specs=[
            pl.BlockSpec((K, w), lambda s: (0, s)),
            pl.BlockSpec((kt, w), lambda s, r=ksc // kt: (r, s)),
        ],
        out_specs=([pl.BlockSpec((w,), lambda s: (s,)) for _ in range(K)]
                   + [pl.BlockSpec((w,), lambda s: (s,)) for _ in range(kt)]),
        out_shape=([jax.ShapeDtypeStruct((M,), utable.dtype) for _ in range(K)]
                   + [jax.ShapeDtypeStruct((N,), itable.dtype) for _ in range(kt)]),
    )(utable, itable)


def kernel(user, item, user_emb, item_emb):
    B = user.shape[0]
    K, N = item_emb.shape
    M = user_emb.shape[0]
    NW = _NC * _NS
    CH = B // NW
    KT = K - _KSC

    uT = user_emb.T
    mesh = plsc.VectorSubcoreMesh(core_axis_name="c", subcore_axis_name="s")
    cp = pltpu.CompilerParams()
    if "needs_layout_passes" in pltpu.CompilerParams.__dataclass_fields__:
        cp = dataclasses.replace(cp, needs_layout_passes=False)
    if "use_tc_tiling_on_sc" in pltpu.CompilerParams.__dataclass_fields__:
        cp = dataclasses.replace(cp, use_tc_tiling_on_sc=True)

    nfull = (N // _PW) * _PW
    nchunk = N // _PW
    tail = N - nfull

    @functools.partial(
        pl.kernel,
        out_type=jax.ShapeDtypeStruct((_KSC * N,), jnp.float32),
        mesh=mesh,
        compiler_params=cp,
        scratch_types=[
            pltpu.VMEM((_NB * _PW,), jnp.float32),
            pltpu.VMEM((_NB * _PW,), jnp.float32),
            pltpu.VMEM((tail,), jnp.float32),
            pltpu.SemaphoreType.DMA,
            pltpu.SemaphoreType.DMA,
            pltpu.SemaphoreType.DMA,
        ],
    )
    def sc_detile(tab_hbm, out_hbm, buf_a, buf_b, tail_v, sem_a, sem_b, sem_t):
        wid = lax.axis_index("s") * _NC + lax.axis_index("c")
        kk = wid

        def read_batch(b0, nb, buf, sem):
            cps = []
            for q in range(nb):
                cps.append(pltpu.async_copy(
                    tab_hbm.at[kk].at[pl.ds((b0 + q) * _PW, _PW)],
                    buf.at[pl.ds(q * _PW, _PW)], sem))
            return cps

        def write_batch(b0, nb, buf, sem):
            cps = []
            for q in range(nb):
                cps.append(pltpu.async_copy(
                    buf.at[pl.ds(q * _PW, _PW)],
                    out_hbm.at[pl.ds(kk * N + (b0 + q) * _PW, _PW)], sem))
            return cps

        nbat = nchunk // _NB
        rem = nchunk - nbat * _NB
        prev_writes = []
        cur_reads = read_batch(0, _NB, buf_a, sem_a)
        for b in range(nbat):
            nxt = b + 1
            use_a = (b % 2) == 0
            buf = buf_a if use_a else buf_b
            nbuf = buf_b if use_a else buf_a
            nsem = sem_b if use_a else sem_a
            for c in cur_reads:
                c.wait()
            if nxt < nbat:
                nxt_reads = read_batch(nxt * _NB, _NB, nbuf, nsem)
            elif rem > 0:
                nxt_reads = read_batch(nbat * _NB, rem, nbuf, nsem)
            else:
                nxt_reads = []
            for c in prev_writes:
                c.wait()
            prev_writes = write_batch(b * _NB, _NB, buf,
                                      sem_a if use_a else sem_b)
            cur_reads = nxt_reads
        if rem > 0:
            buf = buf_a if (nbat % 2) == 0 else buf_b
            for c in cur_reads:
                c.wait()
            for c in prev_writes:
                c.wait()
            prev_writes = write_batch(nbat * _NB, rem, buf,
                                      sem_a if (nbat % 2) == 0 else sem_b)
        if tail > 0:
            pltpu.async_copy(
                tab_hbm.at[kk].at[pl.ds(nfull, tail)], tail_v, sem_t).wait()
            for c in prev_writes:
                c.wait()
            pltpu.async_copy(
                tail_v, out_hbm.at[pl.ds(kk * N + nfull, tail)], sem_t).wait()
        else:
            for c in prev_writes:
                c.wait()

    @functools.partial(
        pl.kernel,
        out_type=jax.ShapeDtypeStruct((B,), jnp.float32),
        mesh=mesh,
        compiler_params=cp,
        scratch_types=[
            pltpu.VMEM((CH,), jnp.int32),
            pltpu.VMEM((CH,), jnp.int32),
            pltpu.VMEM((_KSC * CH,), jnp.int32),
            pltpu.VMEM((K * CH,), jnp.float32),
            pltpu.VMEM((K * CH,), jnp.float32),
            pltpu.VMEM((CH,), jnp.float32),
            pltpu.SemaphoreType.DMA,
            pltpu.SemaphoreType.DMA,
        ],
    )
    def sc_gather(*refs):
        user_hbm, item_hbm, iflat_hbm = refs[0], refs[1], refs[2]
        utabs = refs[3:3 + K]
        itabs = refs[3 + K:3 + K + KT]
        out_hbm = refs[3 + K + KT]
        (uidx_v, iidx_v, gidx_v, u_v, v_v, acc_v,
         sem_u, sem_v) = refs[4 + K + KT:12 + K + KT]

        wid = lax.axis_index("s") * _NC + lax.axis_index("c")
        base = wid * CH
        pltpu.sync_copy(user_hbm.at[pl.ds(base, CH)], uidx_v)
        pltpu.sync_copy(item_hbm.at[pl.ds(base, CH)], iidx_v)

        cps = []
        for j in range(K):
            cps.append(pltpu.async_copy(
                utabs[j].at[uidx_v], u_v.at[pl.ds(j * CH, CH)], sem_u))
        for j in range(KT):
            cps.append(pltpu.async_copy(
                itabs[j].at[iidx_v], v_v.at[pl.ds((_KSC + j) * CH, CH)], sem_v))

        @pl.loop(0, _KSC)
        def _build(kk):
            off = kk * N
            for c in range(CH // _L):
                gidx_v[pl.ds(kk * CH + c * _L, _L)] = (
                    iidx_v[pl.ds(c * _L, _L)] + off)

        cp_v = pltpu.async_copy(
            iflat_hbm.at[gidx_v], v_v.at[pl.ds(0, _KSC * CH)], sem_v)
        for c in cps:
            c.wait()
        cp_v.wait()

        @pl.loop(0, CH, step=_L)
        def _mac(c):
            acc = jnp.zeros((_L,), jnp.float32)
            for j in range(K):
                acc = acc + u_v[pl.ds(j * CH + c, _L)] * v_v[pl.ds(j * CH + c, _L)]
            acc = jnp.maximum(4.0 - acc, 0.0)
            acc = jnp.maximum(4.0 - acc, 0.0) + 1.0
            acc_v[pl.ds(c, _L)] = acc

        pltpu.sync_copy(acc_v, out_hbm.at[pl.ds(base, CH)])

    tabs = _tc_detile(uT, item_emb, _KSC, _W)
    utabs, itabs = tabs[:K], tabs[K:]
    iflat = sc_detile(item_emb)
    return sc_gather(user, item, iflat, *utabs, *itabs)
